# Initial kernel scaffold; baseline (speedup 1.0000x reference)
import jax
import jax.numpy as jnp
from jax.experimental import pallas as pl


def kernel(x, new_weight, orig_weight):
    raise NotImplementedError("write your pallas kernel here")


# speedup vs baseline: 1.0296x; 1.0296x over previous
import jax
import jax.numpy as jnp
from jax.experimental import pallas as pl
from jax.experimental.pallas import tpu as pltpu

_T = 512


def _mm_kernel(x_ref, nw_ref, ow_ref, o1_ref, o2_ref):
    x = x_ref[...]
    dn = (((1,), (1,)), ((), ()))
    o1_ref[...] = jax.lax.dot_general(x, nw_ref[...], dn,
                                      preferred_element_type=jnp.float32)
    o2_ref[...] = jax.lax.dot_general(x, ow_ref[...], dn,
                                      preferred_element_type=jnp.float32)


def kernel(x, new_weight, orig_weight):
    M, K = x.shape
    N = new_weight.shape[0]
    out1, out2 = pl.pallas_call(
        _mm_kernel,
        grid=(N // _T,),
        in_specs=[
            pl.BlockSpec((M, K), lambda j: (0, 0)),
            pl.BlockSpec((_T, K), lambda j: (j, 0)),
            pl.BlockSpec((_T, K), lambda j: (j, 0)),
        ],
        out_specs=[
            pl.BlockSpec((M, _T), lambda j: (0, j)),
            pl.BlockSpec((M, _T), lambda j: (0, j)),
        ],
        out_shape=[
            jax.ShapeDtypeStruct((M, N), jnp.float32),
            jax.ShapeDtypeStruct((M, N), jnp.float32),
        ],
        compiler_params=pltpu.CompilerParams(
            dimension_semantics=("arbitrary",)),
    )(x, new_weight, orig_weight)
    return (out1, out2)


# speedup vs baseline: 1.0662x; 1.0356x over previous
import jax
import jax.numpy as jnp
from jax.experimental import pallas as pl
from jax.experimental.pallas import tpu as pltpu

_T = 256


def _mm_kernel(x_ref, nw_ref, ow_ref, o1_ref, o2_ref):
    x = x_ref[...]
    dn = (((1,), (1,)), ((), ()))
    o1_ref[...] = jax.lax.dot_general(x, nw_ref[...], dn,
                                      preferred_element_type=jnp.float32)
    o2_ref[...] = jax.lax.dot_general(x, ow_ref[...], dn,
                                      preferred_element_type=jnp.float32)


def kernel(x, new_weight, orig_weight):
    M, K = x.shape
    N = new_weight.shape[0]
    out1, out2 = pl.pallas_call(
        _mm_kernel,
        grid=(N // _T,),
        in_specs=[
            pl.BlockSpec((M, K), lambda j: (0, 0)),
            pl.BlockSpec((_T, K), lambda j: (j, 0)),
            pl.BlockSpec((_T, K), lambda j: (j, 0)),
        ],
        out_specs=[
            pl.BlockSpec((M, _T), lambda j: (0, j)),
            pl.BlockSpec((M, _T), lambda j: (0, j)),
        ],
        out_shape=[
            jax.ShapeDtypeStruct((M, N), jnp.float32),
            jax.ShapeDtypeStruct((M, N), jnp.float32),
        ],
        compiler_params=pltpu.CompilerParams(
            dimension_semantics=("arbitrary",)),
    )(x, new_weight, orig_weight)
    return (out1, out2)
